# baseline (speedup 1.0000x reference)
import functools

import jax
import jax.numpy as jnp
from jax import lax
from jax.experimental import pallas as pl
from jax.experimental.pallas import tpu as pltpu
from jax.experimental.pallas import tpu_sc as plsc

_NC = 2
_NS = 16
_NW = _NC * _NS
_LANES = 16
_NBUF = 4


@functools.lru_cache(maxsize=None)
def _build(n_rows: int, d: int, t: int, vocab: int):
    assert n_rows % _NW == 0
    rows_per_w = n_rows // _NW
    chunk = t
    assert rows_per_w % chunk == 0
    n_chunks = rows_per_w // chunk
    assert n_chunks >= 2 * _NBUF and (n_chunks - 4) % _NBUF == 0
    vecs_per_row = d // _LANES

    mesh = plsc.VectorSubcoreMesh(core_axis_name="c", subcore_axis_name="s")

    @functools.partial(
        pl.kernel,
        mesh=mesh,
        compiler_params=pltpu.CompilerParams(use_tc_tiling_on_sc=False),
        out_type=jax.ShapeDtypeStruct((n_rows, d), jnp.float32),
        scratch_types=[
            pltpu.VMEM((rows_per_w,), jnp.int32),
            pltpu.VMEM((t, d), jnp.float32),
        ]
        + [pltpu.VMEM((chunk, d), jnp.float32) for _ in range(_NBUF)]
        + [pltpu.SemaphoreType.DMA((_NBUF,)), pltpu.SemaphoreType.DMA((_NBUF,))],
    )
    def emb_kernel(idx_hbm, table_hbm, pos_hbm, out_hbm,
                   idx_v, pos_v, b0, b1, b2, b3, gsem, ssem):
        bufs = [b0, b1, b2, b3]
        wid = lax.axis_index("s") * _NC + lax.axis_index("c")
        base = wid * rows_per_w
        pltpu.sync_copy(idx_hbm.at[pl.ds(base, rows_per_w)], idx_v)
        pltpu.sync_copy(pos_hbm.at[pl.ds(0, t)], pos_v)

        def gather_desc(g, s):
            return pltpu.make_async_copy(
                table_hbm.at[idx_v.at[pl.ds(g * chunk, chunk)]],
                bufs[s], gsem.at[s])

        def store_desc(g, s):
            return pltpu.make_async_copy(
                bufs[s], out_hbm.at[pl.ds(base + g * chunk, chunk)], ssem.at[s])

        def do_add(s):
            buf = bufs[s]

            def row_body(r, _):
                for u in range(2):
                    for c in range(vecs_per_row):
                        sl = pl.ds(c * _LANES, _LANES)
                        plsc.addupdate(buf.at[2 * r + u, sl], pos_v[2 * r + u, sl])
                return 0

            lax.fori_loop(0, chunk // 2, row_body, 0)

        gather_desc(0, 0).start()
        gather_desc(1, 1).start()
        for g in (0, 1):
            gather_desc(g, g).wait()
            do_add(g)
            store_desc(g, g).start()
            gather_desc(g + 2, g + 2).start()

        def block_body(i, _):
            g0 = 2 + i * _NBUF
            for b in range(_NBUF):
                g = g0 + b
                s = (2 + b) % _NBUF
                gather_desc(g, s).wait()
                do_add(s)
                store_desc(g, s).start()
                s2 = b % _NBUF
                store_desc(g - 2, s2).wait()
                gather_desc(g + 2, s2).start()
            return 0

        lax.fori_loop(0, (n_chunks - 4) // _NBUF, block_body, 0)

        for g in (n_chunks - 2, n_chunks - 1):
            s = g % _NBUF
            gather_desc(g, s).wait()
            do_add(s)
            store_desc(g, s).start()
            store_desc(g - 2, (g - 2) % _NBUF).wait()
        for g in (n_chunks - 2, n_chunks - 1):
            store_desc(g, g % _NBUF).wait()

    return emb_kernel


def kernel(x, emb_table, pos_table):
    b, t = x.shape
    vocab, d = emb_table.shape
    xf = x.reshape(-1).astype(jnp.int32)
    out = _build(b * t, d, t, vocab)(xf, emb_table, pos_table)
    return out.reshape(b, t, d)


# speedup vs baseline: 1.0040x; 1.0040x over previous
import functools

import jax
import jax.numpy as jnp
from jax import lax
from jax.experimental import pallas as pl
from jax.experimental.pallas import tpu as pltpu
from jax.experimental.pallas import tpu_sc as plsc

_NC = 2
_NS = 16
_NW = _NC * _NS
_LANES = 16
_NBUF = 4


@functools.lru_cache(maxsize=None)
def _build(b: int, t: int, d: int, vocab: int):
    assert b % _NW == 0
    rows_per_w = b // _NW
    n_chunks = rows_per_w
    assert n_chunks >= 2 * _NBUF and (n_chunks - 4) % _NBUF == 0
    vecs_per_row = d // _LANES

    mesh = plsc.VectorSubcoreMesh(core_axis_name="c", subcore_axis_name="s")

    @functools.partial(
        pl.kernel,
        mesh=mesh,
        compiler_params=pltpu.CompilerParams(use_tc_tiling_on_sc=False),
        out_type=jax.ShapeDtypeStruct((b, t, d), jnp.float32),
        scratch_types=[
            pltpu.VMEM((rows_per_w, t), jnp.int32),
            pltpu.VMEM((t, d), jnp.float32),
        ]
        + [pltpu.VMEM((t, d), jnp.float32) for _ in range(_NBUF)]
        + [pltpu.SemaphoreType.DMA((_NBUF,)), pltpu.SemaphoreType.DMA((_NBUF,))],
    )
    def emb_kernel(idx_hbm, table_hbm, pos_hbm, out_hbm,
                   idx_v, pos_v, b0, b1, b2, b3, gsem, ssem):
        bufs = [b0, b1, b2, b3]
        wid = lax.axis_index("s") * _NC + lax.axis_index("c")
        base = wid * rows_per_w
        pltpu.sync_copy(idx_hbm.at[pl.ds(base, rows_per_w)], idx_v)
        pltpu.sync_copy(pos_hbm.at[pl.ds(0, t)], pos_v)

        def gather_desc(g, s):
            return pltpu.make_async_copy(
                table_hbm.at[idx_v.at[g]], bufs[s], gsem.at[s])

        def store_desc(g, s):
            return pltpu.make_async_copy(bufs[s], out_hbm.at[base + g], ssem.at[s])

        def do_add(s):
            buf = bufs[s]

            def row_body(r, _):
                for u in range(2):
                    for c in range(vecs_per_row):
                        sl = pl.ds(c * _LANES, _LANES)
                        plsc.addupdate(buf.at[2 * r + u, sl], pos_v[2 * r + u, sl])
                return 0

            lax.fori_loop(0, t // 2, row_body, 0)

        gather_desc(0, 0).start()
        gather_desc(1, 1).start()
        for g in (0, 1):
            gather_desc(g, g).wait()
            do_add(g)
            store_desc(g, g).start()
            gather_desc(g + 2, g + 2).start()

        def block_body(i, _):
            g0 = 2 + i * _NBUF
            for bb in range(_NBUF):
                g = g0 + bb
                s = (2 + bb) % _NBUF
                gather_desc(g, s).wait()
                do_add(s)
                store_desc(g, s).start()
                s2 = bb % _NBUF
                store_desc(g - 2, s2).wait()
                gather_desc(g + 2, s2).start()
            return 0

        lax.fori_loop(0, (n_chunks - 4) // _NBUF, block_body, 0)

        for g in (n_chunks - 2, n_chunks - 1):
            s = g % _NBUF
            gather_desc(g, s).wait()
            do_add(s)
            store_desc(g, s).start()
            store_desc(g - 2, (g - 2) % _NBUF).wait()
        for g in (n_chunks - 2, n_chunks - 1):
            store_desc(g, g % _NBUF).wait()

    return emb_kernel


def kernel(x, emb_table, pos_table):
    b, t = x.shape
    vocab, d = emb_table.shape
    xi = x if x.dtype == jnp.int32 else x.astype(jnp.int32)
    return _build(b, t, d, vocab)(xi, emb_table, pos_table)


# speedup vs baseline: 1.1197x; 1.1152x over previous
import functools

import jax
import jax.numpy as jnp
from jax import lax
from jax.experimental import pallas as pl
from jax.experimental.pallas import tpu as pltpu
from jax.experimental.pallas import tpu_sc as plsc

_NC = 2
_NS = 16
_NW = _NC * _NS
_LANES = 16


@functools.lru_cache(maxsize=None)
def _build(b: int, t: int, d: int, vocab: int, tmax: int):
    assert d % _NW == 0
    passes = d // _NW
    assert b % (2 * _LANES) == 0
    n_vecs = b // _LANES

    mesh = plsc.VectorSubcoreMesh(core_axis_name="c", subcore_axis_name="s")

    @functools.partial(
        pl.kernel,
        mesh=mesh,
        compiler_params=pltpu.CompilerParams(
            use_tc_tiling_on_sc=True, needs_layout_passes=False),
        out_type=jax.ShapeDtypeStruct((t, d, b), jnp.float32),
        scratch_types=[
            pltpu.VMEM((vocab,), jnp.float32),
            pltpu.VMEM((tmax,), jnp.float32),
            pltpu.VMEM((b,), jnp.int32),
            pltpu.VMEM((b,), jnp.int32),
            pltpu.VMEM((b,), jnp.float32),
            pltpu.VMEM((b,), jnp.float32),
            pltpu.SemaphoreType.DMA((2,)),
            pltpu.SemaphoreType.DMA((2,)),
        ],
    )
    def emb_kernel(xt_hbm, tablet_hbm, post_hbm, out_hbm,
                   tab_v, pos_v, i0, i1, o0, o1, isem, osem):
        ibufs = [i0, i1]
        obufs = [o0, o1]
        wid = lax.axis_index("s") * _NC + lax.axis_index("c")

        def idx_desc(tt, s):
            return pltpu.make_async_copy(xt_hbm.at[tt], ibufs[s], isem.at[s])

        def compute(tt, s, c):
            pvec = plsc.load_gather(pos_v, [jnp.full((_LANES,), tt, jnp.int32)])
            ib, ob = ibufs[s], obufs[s]

            def kbody(k, _):
                for u in range(4):
                    sl = pl.ds((4 * k + u) * _LANES, _LANES)
                    ob[sl] = plsc.load_gather(tab_v, [ib[sl]]) + pvec
                return 0

            lax.fori_loop(0, n_vecs // 4, kbody, 0)

        def store_desc(tt, s, c):
            return pltpu.make_async_copy(obufs[s], out_hbm.at[tt, c], osem.at[s])

        for p in range(passes):
            c = wid + _NW * p
            pltpu.sync_copy(tablet_hbm.at[c], tab_v)
            pltpu.sync_copy(post_hbm.at[c], pos_v)

            idx_desc(0, 0).start()
            idx_desc(1, 1).start()

            def tbody(i, _):
                for s in range(2):
                    tt = 2 * i + s

                    @pl.when(tt < t)
                    def _body():
                        idx_desc(tt, s).wait()

                        @pl.when(tt >= 2)
                        def _():
                            store_desc(tt - 2, s, c).wait()

                        compute(tt, s, c)
                        store_desc(tt, s, c).start()

                        @pl.when(tt + 2 < t)
                        def _():
                            idx_desc(tt + 2, s).start()

                    @pl.when(tt >= t)
                    def _drain():
                        store_desc(tt - 2, s, c).wait()
                return 0

            lax.fori_loop(0, (t + 2) // 2, tbody, 0)

    return emb_kernel


def kernel(x, emb_table, pos_table):
    b, t = x.shape
    vocab, d = emb_table.shape
    tmax = pos_table.shape[0]
    xi = x if x.dtype == jnp.int32 else x.astype(jnp.int32)
    vocab_pad = (-vocab) % 128
    table = jnp.pad(emb_table, ((0, vocab_pad), (0, 0))) if vocab_pad else emb_table
    out_t = _build(b, t, d, vocab + vocab_pad, tmax)(xi.T, table.T, pos_table.T)
    return out_t.transpose(2, 0, 1)


# speedup vs baseline: 2.0958x; 1.8718x over previous
import functools

import jax
import jax.numpy as jnp
from jax import lax
from jax.experimental import pallas as pl
from jax.experimental.pallas import tpu as pltpu
from jax.experimental.pallas import tpu_sc as plsc

_NC = 2
_NS = 16
_NW = _NC * _NS
_LANES = 16


@functools.lru_cache(maxsize=None)
def _build(b: int, t: int, d: int, vocab: int, tmax: int):
    assert d % _NW == 0
    passes = d // _NW
    assert b % (2 * _LANES) == 0
    n_vecs = b // _LANES

    mesh = plsc.VectorSubcoreMesh(core_axis_name="c", subcore_axis_name="s")

    @functools.partial(
        pl.kernel,
        mesh=mesh,
        compiler_params=pltpu.CompilerParams(
            use_tc_tiling_on_sc=True, needs_layout_passes=False),
        out_type=jax.ShapeDtypeStruct((t, d, b), jnp.float32),
        scratch_types=[
            pltpu.VMEM((vocab,), jnp.float32),
            pltpu.VMEM((tmax,), jnp.float32),
            pltpu.VMEM((b,), jnp.int32),
            pltpu.VMEM((b,), jnp.int32),
            pltpu.VMEM((b,), jnp.float32),
            pltpu.VMEM((b,), jnp.float32),
            pltpu.SemaphoreType.DMA((2,)),
            pltpu.SemaphoreType.DMA((2,)),
        ],
    )
    def emb_kernel(xt_hbm, tablet_hbm, post_hbm, out_hbm,
                   tab_v, pos_v, i0, i1, o0, o1, isem, osem):
        ibufs = [i0, i1]
        obufs = [o0, o1]
        wid = lax.axis_index("s") * _NC + lax.axis_index("c")

        def idx_desc(tt, s):
            return pltpu.make_async_copy(xt_hbm.at[tt], ibufs[s], isem.at[s])

        def compute(tt, s, c):
            pvec = plsc.load_gather(pos_v, [jnp.full((_LANES,), tt, jnp.int32)])
            ib, ob = ibufs[s], obufs[s]

            def kbody(k, _):
                sls = [pl.ds((8 * k + u) * _LANES, _LANES) for u in range(8)]
                idxs = [ib[sl] for sl in sls]
                gats = [plsc.load_gather(tab_v, [idx]) for idx in idxs]
                for sl, g in zip(sls, gats):
                    ob[sl] = g + pvec
                return 0

            lax.fori_loop(0, n_vecs // 8, kbody, 0)

        def store_desc(tt, s, c):
            return pltpu.make_async_copy(obufs[s], out_hbm.at[tt, c], osem.at[s])

        for p in range(passes):
            c = wid + _NW * p
            pltpu.sync_copy(tablet_hbm.at[c], tab_v)
            pltpu.sync_copy(post_hbm.at[c], pos_v)

            idx_desc(0, 0).start()
            idx_desc(1, 1).start()

            def tbody(i, _):
                for s in range(2):
                    tt = 2 * i + s

                    @pl.when(tt < t)
                    def _body():
                        idx_desc(tt, s).wait()

                        @pl.when(tt >= 2)
                        def _():
                            store_desc(tt - 2, s, c).wait()

                        compute(tt, s, c)
                        store_desc(tt, s, c).start()

                        @pl.when(tt + 2 < t)
                        def _():
                            idx_desc(tt + 2, s).start()

                    @pl.when(tt >= t)
                    def _drain():
                        store_desc(tt - 2, s, c).wait()
                return 0

            lax.fori_loop(0, (t + 2) // 2, tbody, 0)

    return emb_kernel


def kernel(x, emb_table, pos_table):
    b, t = x.shape
    vocab, d = emb_table.shape
    tmax = pos_table.shape[0]
    xi = x if x.dtype == jnp.int32 else x.astype(jnp.int32)
    vocab_pad = (-vocab) % 128
    table = jnp.pad(emb_table, ((0, vocab_pad), (0, 0))) if vocab_pad else emb_table
    out_t = _build(b, t, d, vocab + vocab_pad, tmax)(xi.T, table.T, pos_table.T)
    return out_t.transpose(2, 0, 1)


# speedup vs baseline: 2.1841x; 1.0421x over previous
import functools

import jax
import jax.numpy as jnp
from jax import lax
from jax.experimental import pallas as pl
from jax.experimental.pallas import tpu as pltpu
from jax.experimental.pallas import tpu_sc as plsc

_NC = 2
_NS = 16
_NW = _NC * _NS
_LANES = 16


@functools.lru_cache(maxsize=None)
def _build(b: int, t: int, d: int, vocab: int, tmax: int):
    assert d % _NW == 0
    passes = d // _NW
    assert b % (2 * _LANES) == 0
    n_vecs = b // _LANES

    mesh = plsc.VectorSubcoreMesh(core_axis_name="c", subcore_axis_name="s")

    @functools.partial(
        pl.kernel,
        mesh=mesh,
        compiler_params=pltpu.CompilerParams(
            use_tc_tiling_on_sc=True, needs_layout_passes=False),
        out_type=jax.ShapeDtypeStruct((t, d, b), jnp.float32),
        scratch_types=[
            pltpu.VMEM((vocab,), jnp.float32),
            pltpu.VMEM((tmax,), jnp.float32),
            pltpu.VMEM((b,), jnp.int32),
            pltpu.VMEM((b,), jnp.int32),
            pltpu.VMEM((b,), jnp.float32),
            pltpu.VMEM((b,), jnp.float32),
            pltpu.SemaphoreType.DMA((2,)),
            pltpu.SemaphoreType.DMA((2,)),
        ],
    )
    def emb_kernel(xt_hbm, tablet_hbm, post_hbm, out_hbm,
                   tab_v, pos_v, i0, i1, o0, o1, isem, osem):
        ibufs = [i0, i1]
        obufs = [o0, o1]
        wid = lax.axis_index("s") * _NC + lax.axis_index("c")

        def idx_desc(tt, s):
            return pltpu.make_async_copy(xt_hbm.at[tt], ibufs[s], isem.at[s])

        def compute(tt, s, c):
            pvec = plsc.load_gather(pos_v, [jnp.full((_LANES,), tt, jnp.int32)])
            ib, ob = ibufs[s], obufs[s]

            def kbody(k, _):
                sls = [pl.ds((8 * k + u) * _LANES, _LANES) for u in range(8)]
                idxs = [ib[sl] for sl in sls]
                gats = [plsc.load_gather(tab_v, [idx]) for idx in idxs]
                for sl, g in zip(sls, gats):
                    ob[sl] = g + pvec
                return 0

            lax.fori_loop(0, n_vecs // 8, kbody, 0)

        def store_desc(tt, s, c):
            return pltpu.make_async_copy(obufs[s], out_hbm.at[tt, c], osem.at[s])

        for p in range(passes):
            c = wid + _NW * p
            pltpu.sync_copy(tablet_hbm.at[c], tab_v)
            pltpu.sync_copy(post_hbm.at[c], pos_v)

            idx_desc(0, 0).start()
            idx_desc(1, 1).start()

            def tbody(i, _):
                for s in range(2):
                    tt = 2 * i + s

                    @pl.when(tt < t)
                    def _body():
                        idx_desc(tt, s).wait()

                        @pl.when(tt >= 2)
                        def _():
                            store_desc(tt - 2, s, c).wait()

                        compute(tt, s, c)
                        store_desc(tt, s, c).start()

                        @pl.when(tt + 2 < t)
                        def _():
                            idx_desc(tt + 2, s).start()

                    @pl.when(tt >= t)
                    def _drain():
                        store_desc(tt - 2, s, c).wait()
                return 0

            lax.fori_loop(0, (t + 2) // 2, tbody, 0)

    return emb_kernel


def kernel(x, emb_table, pos_table):
    b, t = x.shape
    vocab, d = emb_table.shape
    tmax = pos_table.shape[0]
    xi = x if x.dtype == jnp.int32 else x.astype(jnp.int32)
    out_t = _build(b, t, d, vocab, tmax)(xi.T, emb_table.T, pos_table.T)
    return out_t.transpose(2, 0, 1)


# speedup vs baseline: 2.1899x; 1.0027x over previous
import functools

import jax
import jax.numpy as jnp
from jax import lax
from jax.experimental import pallas as pl
from jax.experimental.pallas import tpu as pltpu
from jax.experimental.pallas import tpu_sc as plsc

_NC = 2
_NS = 16
_NW = _NC * _NS
_LANES = 16


@functools.lru_cache(maxsize=None)
def _build(b: int, t: int, d: int, vocab: int, tmax: int):
    assert d % _NW == 0
    passes = d // _NW
    assert b % (2 * _LANES) == 0
    n_vecs = b // _LANES

    mesh = plsc.VectorSubcoreMesh(core_axis_name="c", subcore_axis_name="s")

    @functools.partial(
        pl.kernel,
        mesh=mesh,
        compiler_params=pltpu.CompilerParams(
            use_tc_tiling_on_sc=True, needs_layout_passes=False),
        out_type=jax.ShapeDtypeStruct((t, d, b), jnp.float32),
        scratch_types=[
            pltpu.VMEM((vocab,), jnp.float32),
            pltpu.VMEM((tmax,), jnp.float32),
            pltpu.VMEM((b,), jnp.int32),
            pltpu.VMEM((b,), jnp.int32),
            pltpu.VMEM((b,), jnp.float32),
            pltpu.VMEM((b,), jnp.float32),
            pltpu.SemaphoreType.DMA((2,)),
            pltpu.SemaphoreType.DMA((2,)),
        ],
    )
    def emb_kernel(xt_hbm, tablet_hbm, post_hbm, out_hbm,
                   tab_v, pos_v, i0, i1, o0, o1, isem, osem):
        ibufs = [i0, i1]
        obufs = [o0, o1]
        wid = lax.axis_index("s") * _NC + lax.axis_index("c")

        def idx_desc(tt, s):
            return pltpu.make_async_copy(xt_hbm.at[tt], ibufs[s], isem.at[s])

        def compute(tt, s, c):
            pvec = plsc.load_gather(pos_v, [jnp.full((_LANES,), tt, jnp.int32)])
            ib, ob = ibufs[s], obufs[s]

            def kbody(k, _):
                sls = [pl.ds((16 * k + u) * _LANES, _LANES) for u in range(16)]
                idxs = [ib[sl] for sl in sls]
                gats = [plsc.load_gather(tab_v, [idx]) for idx in idxs]
                for sl, g in zip(sls, gats):
                    ob[sl] = g + pvec
                return 0

            lax.fori_loop(0, n_vecs // 16, kbody, 0)

        def store_desc(tt, s, c):
            return pltpu.make_async_copy(obufs[s], out_hbm.at[tt, c], osem.at[s])

        for p in range(passes):
            c = wid + _NW * p
            pltpu.sync_copy(tablet_hbm.at[c], tab_v)
            pltpu.sync_copy(post_hbm.at[c], pos_v)

            idx_desc(0, 0).start()
            idx_desc(1, 1).start()

            def tbody(i, _):
                for s in range(2):
                    tt = 2 * i + s

                    @pl.when(tt < t)
                    def _body():
                        idx_desc(tt, s).wait()

                        @pl.when(tt >= 2)
                        def _():
                            store_desc(tt - 2, s, c).wait()

                        compute(tt, s, c)
                        store_desc(tt, s, c).start()

                        @pl.when(tt + 2 < t)
                        def _():
                            idx_desc(tt + 2, s).start()

                    @pl.when(tt >= t)
                    def _drain():
                        store_desc(tt - 2, s, c).wait()
                return 0

            lax.fori_loop(0, (t + 2) // 2, tbody, 0)

    return emb_kernel


def kernel(x, emb_table, pos_table):
    b, t = x.shape
    vocab, d = emb_table.shape
    tmax = pos_table.shape[0]
    xi = x if x.dtype == jnp.int32 else x.astype(jnp.int32)
    out_t = _build(b, t, d, vocab, tmax)(xi.T, emb_table.T, pos_table.T)
    return out_t.transpose(2, 0, 1)
